# baseline (speedup 1.0000x reference)
import functools
import math

import jax
import jax.numpy as jnp
from jax import lax
from jax.experimental import pallas as pl
from jax.experimental.pallas import tpu as pltpu
from jax.experimental.pallas import tpu_sc as plsc

_EPS = 1e-07
_BATCH = 1024
_FEAT = 128
_NDATA = 1000000
_K1 = 512
_T = 0.07

_NC = 2
_NS = 16
_NW = _NC * _NS
_BPW = _BATCH // _NW
_CH = 128
_DEPTH = 4
_NCH = _K1 // _CH
_L = 16


def _embed_body(x_ref, w_ref, b_ref, o_ref):
    x = x_ref[...]
    w = w_ref[...]
    y = lax.dot_general(x, w, (((1,), (1,)), ((), ())),
                        preferred_element_type=jnp.float32)
    y = y + b_ref[...]
    norm = jnp.sqrt(jnp.sum(y * y, axis=1, keepdims=True))
    o_ref[...] = y / norm


def _embed(x, w, b):
    bsz, d = x.shape
    bb = 256
    return pl.pallas_call(
        _embed_body,
        grid=(bsz // bb,),
        in_specs=[
            pl.BlockSpec((bb, d), lambda i: (i, 0)),
            pl.BlockSpec((_FEAT, d), lambda i: (0, 0)),
            pl.BlockSpec((1, _FEAT), lambda i: (0, 0)),
        ],
        out_specs=pl.BlockSpec((bb, _FEAT), lambda i: (i, 0)),
        out_shape=jax.ShapeDtypeStruct((bsz, _FEAT), jnp.float32),
    )(x, w, b.reshape(1, _FEAT))


def _tree_sum(terms):
    while len(terms) > 1:
        nxt = [terms[i] + terms[i + 1] for i in range(0, len(terms) - 1, 2)]
        if len(terms) % 2:
            nxt.append(terms[-1])
        terms = nxt
    return terms[0]


def _merge(accs, lane, d):
    while len(accs) > 1:
        sel = (lane & d) != 0
        nxt = []
        for i in range(0, len(accs), 2):
            a, b = accs[i], accs[i + 1]
            ar = a + a.at[lane ^ d].get(mode="promise_in_bounds")
            br = b + b.at[lane ^ d].get(mode="promise_in_bounds")
            nxt.append(jnp.where(sel, br, ar))
        accs = nxt
        d *= 2
    return accs[0]


def _sc_dots_body(mem1, mem2, cidx, v1, v2, s1o, s2o,
                  idx_v, rows_v, v1_v, v2_v, s1_v, s2_v, sem_arr):
    wid = lax.axis_index("s") * _NC + lax.axis_index("c")
    base = wid * _BPW
    lane = lax.iota(jnp.int32, _L)
    sems = [sem_arr.at[i] for i in range(_DEPTH)]
    banks = (mem2, mem1)
    nseq = 2 * _NCH

    pltpu.sync_copy(v1.at[pl.ds(base, _BPW)], v1_v)
    pltpu.sync_copy(v2.at[pl.ds(base, _BPW)], v2_v)
    pltpu.sync_copy(cidx.at[pl.ds(base, _BPW)], idx_v)

    def gather(bl, seq, buf):
        return pltpu.async_copy(
            banks[seq % 2].at[idx_v.at[bl, pl.ds((seq // 2) * _CH, _CH)]],
            rows_v.at[buf], sems[buf])

    for s in range(_DEPTH - 1):
        gather(0, s, s % _DEPTH)

    def b_loop(bl, carry):
        v1c = [v1_v[bl, pl.ds(g * _L, _L)] for g in range(_FEAT // _L)]
        v2c = [v2_v[bl, pl.ds(g * _L, _L)] for g in range(_FEAT // _L)]
        for seq in range(nseq):
            bank, c, buf = seq % 2, seq // 2, seq % _DEPTH
            fs = seq + _DEPTH - 1
            if fs < nseq:
                gather(bl, fs, fs % _DEPTH)
            else:
                gather(jnp.minimum(bl + 1, _BPW - 1), fs - nseq, fs % _DEPTH)
            pltpu.make_async_copy(
                banks[bank].at[idx_v.at[bl, pl.ds(c * _CH, _CH)]],
                rows_v.at[buf], sems[buf]).wait()
            vc = v1c if bank == 0 else v2c
            s_v = s1_v if bank == 0 else s2_v

            @plsc.parallel_loop(0, _CH // _L, 1, unroll=2)
            def grp(jj):
                octs, pending = [], None
                for q in range(4):
                    accs = []
                    for t4 in range(4):
                        j = jj * _L + q * 4 + t4
                        accs.append(_tree_sum([
                            rows_v[buf, j, pl.ds(g * _L, _L)] * vc[g]
                            for g in range(_FEAT // _L)]))
                    qv = _merge(accs, lane, 1)
                    if pending is None:
                        pending = qv
                    else:
                        octs.append(_merge([pending, qv], lane, 4))
                        pending = None
                s_v[bl, pl.ds(c * _CH + jj * _L, _L)] = _merge(octs, lane, 8)
        return carry

    lax.fori_loop(0, _BPW, b_loop, 0)
    for s in range(_DEPTH - 1):
        pltpu.make_async_copy(
            banks[s % 2].at[idx_v.at[_BPW - 1, pl.ds((s // 2) * _CH, _CH)]],
            rows_v.at[s % _DEPTH], sems[s % _DEPTH]).wait()
    pltpu.sync_copy(s1_v, s1o.at[pl.ds(base, _BPW)])
    pltpu.sync_copy(s2_v, s2o.at[pl.ds(base, _BPW)])


_sc_dots = functools.partial(
    pl.kernel,
    out_type=(jax.ShapeDtypeStruct((_BATCH, _K1), jnp.float32),
              jax.ShapeDtypeStruct((_BATCH, _K1), jnp.float32)),
    mesh=plsc.VectorSubcoreMesh(core_axis_name="c", subcore_axis_name="s"),
    scratch_types=[
        pltpu.VMEM((_BPW, _K1), jnp.int32),
        pltpu.VMEM((_DEPTH, _CH, _FEAT), jnp.float32),
        pltpu.VMEM((_BPW, _FEAT), jnp.float32),
        pltpu.VMEM((_BPW, _FEAT), jnp.float32),
        pltpu.VMEM((_BPW, _K1), jnp.float32),
        pltpu.VMEM((_BPW, _K1), jnp.float32),
        pltpu.SemaphoreType.DMA((_DEPTH,)),
    ],
)(_sc_dots_body)


def _loss_body(s1_ref, s2_ref, o_ref):
    m = float(_K1 - 1)
    pn = 1.0 / float(_NDATA)
    mpn = m * pn

    def side(s_ref):
        e = jnp.exp(s_ref[...] * (1.0 / _T))
        z = jnp.mean(e) * float(_NDATA)
        p = e / z
        col0 = p[:, 0:1]
        log_d1 = jnp.log(col0 / (col0 + mpn + _EPS))
        log_d0_all = jnp.log(mpn / (p + mpn + _EPS))
        log_d0_col0 = jnp.log(mpn / (col0 + mpn + _EPS))
        return -(jnp.sum(log_d1) + jnp.sum(log_d0_all)
                 - jnp.sum(log_d0_col0)) / float(_BATCH)

    o_ref[0, 0] = side(s1_ref) + side(s2_ref)


def _loss(s1, s2):
    return pl.pallas_call(
        _loss_body,
        in_specs=[
            pl.BlockSpec((_BATCH, _K1), lambda: (0, 0)),
            pl.BlockSpec((_BATCH, _K1), lambda: (0, 0)),
        ],
        out_specs=pl.BlockSpec(memory_space=pltpu.MemorySpace.SMEM),
        out_shape=jax.ShapeDtypeStruct((1, 1), jnp.float32),
    )(s1, s2)


def kernel(f_s, f_t, idx, contrast_idx, W_s, b_s, W_t, b_t,
           memory_v1, memory_v2):
    del idx
    v1 = _embed(f_s, W_s, b_s)
    v2 = _embed(f_t, W_t, b_t)
    s1, s2 = _sc_dots(memory_v1, memory_v2, contrast_idx, v1, v2)
    return _loss(s1, s2).reshape((1,))


# speedup vs baseline: 1.0692x; 1.0692x over previous
import functools
import math

import jax
import jax.numpy as jnp
from jax import lax
from jax.experimental import pallas as pl
from jax.experimental.pallas import tpu as pltpu
from jax.experimental.pallas import tpu_sc as plsc

_EPS = 1e-07
_BATCH = 1024
_FEAT = 128
_NDATA = 1000000
_K1 = 512
_T = 0.07

_NC = 2
_NS = 16
_NW = _NC * _NS
_BPW = _BATCH // _NW
_CH = 128
_DEPTH = 4
_NCH = _K1 // _CH
_L = 16


def _embed_body(x_ref, w_ref, b_ref, o_ref):
    x = x_ref[...]
    w = w_ref[...]
    y = lax.dot_general(x, w, (((1,), (1,)), ((), ())),
                        preferred_element_type=jnp.float32)
    y = y + b_ref[...]
    norm = jnp.sqrt(jnp.sum(y * y, axis=1, keepdims=True))
    o_ref[...] = y / norm


def _embed(x, w, b):
    bsz, d = x.shape
    bb = 256
    return pl.pallas_call(
        _embed_body,
        grid=(bsz // bb,),
        in_specs=[
            pl.BlockSpec((bb, d), lambda i: (i, 0)),
            pl.BlockSpec((_FEAT, d), lambda i: (0, 0)),
            pl.BlockSpec((1, _FEAT), lambda i: (0, 0)),
        ],
        out_specs=pl.BlockSpec((bb, _FEAT), lambda i: (i, 0)),
        out_shape=jax.ShapeDtypeStruct((bsz, _FEAT), jnp.float32),
    )(x, w, b.reshape(1, _FEAT))


def _tree_sum(terms):
    while len(terms) > 1:
        nxt = [terms[i] + terms[i + 1] for i in range(0, len(terms) - 1, 2)]
        if len(terms) % 2:
            nxt.append(terms[-1])
        terms = nxt
    return terms[0]


def _merge(accs, lane, d):
    while len(accs) > 1:
        sel = (lane & d) != 0
        nxt = []
        for i in range(0, len(accs), 2):
            a, b = accs[i], accs[i + 1]
            ar = a + a.at[lane ^ d].get(mode="promise_in_bounds")
            br = b + b.at[lane ^ d].get(mode="promise_in_bounds")
            nxt.append(jnp.where(sel, br, ar))
        accs = nxt
        d *= 2
    return accs[0]


def _sc_dots_body(mem1, mem2, cidx, v1, v2, s1o, s2o,
                  idx_v, rows_v, v1_v, v2_v, s1_v, s2_v, sem_arr):
    wid = lax.axis_index("s") * _NC + lax.axis_index("c")
    base = wid * _BPW
    lane = lax.iota(jnp.int32, _L)
    sems = [sem_arr.at[i] for i in range(_DEPTH)]
    banks = (mem2, mem1)
    nseq = 2 * _NCH

    pltpu.sync_copy(v1.at[pl.ds(base, _BPW)], v1_v)
    pltpu.sync_copy(v2.at[pl.ds(base, _BPW)], v2_v)
    pltpu.sync_copy(cidx.at[pl.ds(base, _BPW)], idx_v)

    def gather(bl, seq, buf):
        return pltpu.async_copy(
            banks[seq % 2].at[idx_v.at[bl, pl.ds((seq // 2) * _CH, _CH)]],
            rows_v.at[buf], sems[buf])

    for s in range(_DEPTH - 1):
        gather(0, s, s % _DEPTH)

    def b_loop(bl, carry):
        v1c = [v1_v[bl, pl.ds(g * _L, _L)] for g in range(_FEAT // _L)]
        v2c = [v2_v[bl, pl.ds(g * _L, _L)] for g in range(_FEAT // _L)]
        for seq in range(nseq):
            bank, c, buf = seq % 2, seq // 2, seq % _DEPTH
            fs = seq + _DEPTH - 1
            if fs < nseq:
                gather(bl, fs, fs % _DEPTH)
            else:
                gather(jnp.minimum(bl + 1, _BPW - 1), fs - nseq, fs % _DEPTH)
            pltpu.make_async_copy(
                banks[bank].at[idx_v.at[bl, pl.ds(c * _CH, _CH)]],
                rows_v.at[buf], sems[buf]).wait()
            vc = v1c if bank == 0 else v2c
            s_v = s1_v if bank == 0 else s2_v

            @plsc.parallel_loop(0, _CH // _L, 1, unroll=2)
            def grp(jj):
                quads = []
                for q in range(4):
                    accs = []
                    for t4 in range(4):
                        j = jj * _L + q * 4 + t4
                        accs.append(_tree_sum([
                            rows_v[buf, j, pl.ds(g * _L, _L)] * vc[g]
                            for g in range(_FEAT // _L)]))
                    quads.append(_merge(accs, lane, 1))
                s_v[bl, pl.ds(c * _CH + jj * _L, _L)] = _merge(quads, lane, 4)
        return carry

    lax.fori_loop(0, _BPW, b_loop, 0)
    for s in range(_DEPTH - 1):
        pltpu.make_async_copy(
            banks[s % 2].at[idx_v.at[_BPW - 1, pl.ds((s // 2) * _CH, _CH)]],
            rows_v.at[s % _DEPTH], sems[s % _DEPTH]).wait()
    pltpu.sync_copy(s1_v, s1o.at[pl.ds(base, _BPW)])
    pltpu.sync_copy(s2_v, s2o.at[pl.ds(base, _BPW)])


_sc_dots = functools.partial(
    pl.kernel,
    out_type=(jax.ShapeDtypeStruct((_BATCH, _K1), jnp.float32),
              jax.ShapeDtypeStruct((_BATCH, _K1), jnp.float32)),
    mesh=plsc.VectorSubcoreMesh(core_axis_name="c", subcore_axis_name="s"),
    scratch_types=[
        pltpu.VMEM((_BPW, _K1), jnp.int32),
        pltpu.VMEM((_DEPTH, _CH, _FEAT), jnp.float32),
        pltpu.VMEM((_BPW, _FEAT), jnp.float32),
        pltpu.VMEM((_BPW, _FEAT), jnp.float32),
        pltpu.VMEM((_BPW, _K1), jnp.float32),
        pltpu.VMEM((_BPW, _K1), jnp.float32),
        pltpu.SemaphoreType.DMA((_DEPTH,)),
    ],
)(_sc_dots_body)


def _loss_body(s1_ref, s2_ref, o_ref):
    m = float(_K1 - 1)
    pn = 1.0 / float(_NDATA)
    mpn = m * pn

    def side(s_ref):
        e = jnp.exp(s_ref[...] * (1.0 / _T))
        z = jnp.mean(e) * float(_NDATA)
        p = e / z
        col0 = p[:, 0:1]
        log_d1 = jnp.log(col0 / (col0 + mpn + _EPS))
        log_d0_all = jnp.log(mpn / (p + mpn + _EPS))
        log_d0_col0 = jnp.log(mpn / (col0 + mpn + _EPS))
        return -(jnp.sum(log_d1) + jnp.sum(log_d0_all)
                 - jnp.sum(log_d0_col0)) / float(_BATCH)

    o_ref[0, 0] = side(s1_ref) + side(s2_ref)


def _loss(s1, s2):
    return pl.pallas_call(
        _loss_body,
        in_specs=[
            pl.BlockSpec((_BATCH, _K1), lambda: (0, 0)),
            pl.BlockSpec((_BATCH, _K1), lambda: (0, 0)),
        ],
        out_specs=pl.BlockSpec(memory_space=pltpu.MemorySpace.SMEM),
        out_shape=jax.ShapeDtypeStruct((1, 1), jnp.float32),
    )(s1, s2)


def kernel(f_s, f_t, idx, contrast_idx, W_s, b_s, W_t, b_t,
           memory_v1, memory_v2):
    del idx
    v1 = _embed(f_s, W_s, b_s)
    v2 = _embed(f_t, W_t, b_t)
    s1, s2 = _sc_dots(memory_v1, memory_v2, contrast_idx, v1, v2)
    return _loss(s1, s2).reshape((1,))
